# Initial kernel scaffold; baseline (speedup 1.0000x reference)
import jax
import jax.numpy as jnp
from jax.experimental import pallas as pl


def kernel(x, edge_index, edge_attr, Wl1, Wr1, We1, att1, b1, bn1_g, bn1_b, Wl2, Wr2, We2, att2, b2, bn2_g, bn2_b, prelu_w):
    raise NotImplementedError("write your pallas kernel here")


# speedup vs baseline: 5.5206x; 5.5206x over previous
import functools

import jax
import jax.numpy as jnp
from jax import lax
from jax.experimental import pallas as pl
from jax.experimental.pallas import tpu as pltpu
from jax.experimental.pallas import tpu_sc as plsc

N = 10000
E = 160000
F = 1024
H = 4
HD = 256
EDIM = 16
NC = 2
NS = 16
NW = NC * NS
EPW = E // NW
EPS = E // NS
EPAD = 5120
EPADL = NW * EPAD
KG = 40
KV = 80
NACC = N * H
FH = F // 2
CH = HD // 2
BR = 312


def _mesh():
    return plsc.VectorSubcoreMesh(
        core_axis_name="c", subcore_axis_name="s", num_cores=NC, num_subcores=NS)


def _mm_body(x_ref, wl_ref, wr_ref, xl_ref, xr_ref):
    x = x_ref[...]
    xl_ref[...] = jnp.dot(x, wl_ref[...], preferred_element_type=jnp.float32)
    xr_ref[...] = jnp.dot(x, wr_ref[...], preferred_element_type=jnp.float32)


def _mm(x, wl, wr):
    n, d = x.shape
    bn = 1000
    return pl.pallas_call(
        _mm_body,
        grid=(n // bn,),
        in_specs=[
            pl.BlockSpec((bn, d), lambda i: (i, 0)),
            pl.BlockSpec((d, F), lambda i: (0, 0)),
            pl.BlockSpec((d, F), lambda i: (0, 0)),
        ],
        out_specs=[
            pl.BlockSpec((bn, F), lambda i: (i, 0)),
            pl.BlockSpec((bn, F), lambda i: (i, 0)),
        ],
        out_shape=[
            jax.ShapeDtypeStruct((n, F), jnp.float32),
            jax.ShapeDtypeStruct((n, F), jnp.float32),
        ],
    )(x, wl, wr)


def _edge_body(gs_ref, gd_ref, ea_ref, we_ref, att_ref, ex_ref, v_ref):
    gs = gs_ref[...]
    u = gs + gd_ref[...] + jnp.dot(ea_ref[...], we_ref[...],
                                   preferred_element_type=jnp.float32)
    m = jnp.maximum(u, 0.0) + 0.2 * jnp.minimum(u, 0.0)
    am = m * att_ref[...]
    for h in range(H):
        sl = slice(h * HD, (h + 1) * HD)
        lg = jnp.sum(am[:, sl], axis=1)
        exh = jnp.exp(lg)
        ex_ref[h, :] = exh
        v_ref[:, sl] = gs[:, sl] * exh[:, None]


def _edge(gs, gd, ea, we, attf):
    be = 1280
    return pl.pallas_call(
        _edge_body,
        grid=(E // be,),
        in_specs=[
            pl.BlockSpec((be, F), lambda i: (i, 0)),
            pl.BlockSpec((be, F), lambda i: (i, 0)),
            pl.BlockSpec((be, EDIM), lambda i: (i, 0)),
            pl.BlockSpec((EDIM, F), lambda i: (0, 0)),
            pl.BlockSpec((1, F), lambda i: (0, 0)),
        ],
        out_specs=[
            pl.BlockSpec((H, be), lambda i: (0, i)),
            pl.BlockSpec((be, F), lambda i: (i, 0)),
        ],
        out_shape=[
            jax.ShapeDtypeStruct((H, E), jnp.float32),
            jax.ShapeDtypeStruct((E, F), jnp.float32),
        ],
    )(gs, gd, ea, we, attf)


def _inv_body(part_ref, inv_ref):
    s = jnp.sum(part_ref[...], axis=0)
    inv_ref[...] = 1.0 / (s + 1e-16)


def _inv(part):
    return pl.pallas_call(
        _inv_body,
        out_shape=jax.ShapeDtypeStruct((NACC,), jnp.float32),
    )(part)


def _post(a0, a1, b, g, bb, pw):
    def body(*refs):
        if pw is None:
            a0_ref, a1_ref, b_ref, g_ref, bb_ref, out_ref = refs
        else:
            a0_ref, a1_ref, b_ref, g_ref, bb_ref, pw_ref, out_ref = refs
        h = (jnp.concatenate([a0_ref[...], a1_ref[...]], axis=1) * 0.25
             + b_ref[...])
        mu = jnp.mean(h, axis=0, keepdims=True)
        d = h - mu
        var = jnp.mean(d * d, axis=0, keepdims=True)
        hn = d * lax.rsqrt(var + 1e-5) * g_ref[...] + bb_ref[...]
        if pw is None:
            out_ref[...] = hn
        else:
            out_ref[...] = (jnp.maximum(hn, 0.0)
                            + pw_ref[...] * jnp.minimum(hn, 0.0))

    args = (a0, a1, b, g, bb) if pw is None else (a0, a1, b, g, bb, pw)
    return pl.pallas_call(
        body,
        out_shape=jax.ShapeDtypeStruct((N, HD), jnp.float32),
    )(*args)


def _sc_gather_body(xl_hbm, xr_hbm, src_hbm, dst_hbm, gs_hbm, gd_hbm,
                    sidx, didx, rl, rr, sem):
    wid = lax.axis_index("s") * NC + lax.axis_index("c")
    base = wid * EPW

    def step(j, carry):
        off = pl.multiple_of(base + j * KG, 8)
        pltpu.sync_copy(src_hbm.at[pl.ds(off, KG)], sidx)
        pltpu.sync_copy(dst_hbm.at[pl.ds(off, KG)], didx)
        pltpu.async_copy(xl_hbm.at[sidx], rl, sem).wait()
        pltpu.async_copy(xr_hbm.at[didx], rr, sem).wait()
        pltpu.sync_copy(rl, gs_hbm.at[pl.ds(off, KG), :])
        pltpu.sync_copy(rr, gd_hbm.at[pl.ds(off, KG), :])
        return carry

    lax.fori_loop(0, EPW // KG, step, 0)


def _sc_gather(xl, xr, src, dst):
    kfn = pl.kernel(
        _sc_gather_body,
        out_type=[
            jax.ShapeDtypeStruct((E, F), jnp.float32),
            jax.ShapeDtypeStruct((E, F), jnp.float32),
        ],
        mesh=_mesh(),
        compiler_params=pltpu.CompilerParams(needs_layout_passes=False),
        scratch_types=[
            pltpu.VMEM((KG,), jnp.int32),
            pltpu.VMEM((KG,), jnp.int32),
            pltpu.VMEM((KG, F), jnp.float32),
            pltpu.VMEM((KG, F), jnp.float32),
            pltpu.SemaphoreType.DMA,
        ],
    )
    return kfn(xl, xr, src, dst)


def _sc_denom_body(dst_hbm, ex_hbm, part_hbm, acc, dbuf, ebuf):
    wid = lax.axis_index("s") * NC + lax.axis_index("c")

    def zero(k, carry):
        acc[pl.ds(pl.multiple_of(k * 16, 8), 16)] = jnp.zeros((16,), jnp.float32)
        return carry

    lax.fori_loop(0, NACC // 16, zero, 0)

    off0 = pl.multiple_of(wid * EPAD, 128)
    pltpu.sync_copy(dst_hbm.at[pl.ds(off0, EPAD)], dbuf)
    pltpu.sync_copy(ex_hbm.at[:, pl.ds(off0, EPAD)], ebuf)

    def step(i, carry):
        off = pl.multiple_of(i * 16, 8)
        dvec = dbuf[pl.ds(off, 16)]
        fours = dvec * H
        for h in range(H):
            plsc.addupdate_scatter(acc, [fours + h], ebuf[h, pl.ds(off, 16)])
        return carry

    lax.fori_loop(0, EPAD // 16, step, 0)
    pltpu.sync_copy(acc, part_hbm.at[wid])


def _sc_denom(dstp, ex4p):
    kfn = pl.kernel(
        _sc_denom_body,
        out_type=jax.ShapeDtypeStruct((NW, NACC), jnp.float32),
        mesh=_mesh(),
        compiler_params=pltpu.CompilerParams(needs_layout_passes=False),
        scratch_types=[
            pltpu.VMEM((NACC,), jnp.float32),
            pltpu.VMEM((EPAD,), jnp.int32),
            pltpu.VMEM((H, EPAD), jnp.float32),
        ],
    )
    return kfn(dstp, ex4p)


def _sc_agg_body(v4_hbm, dst_hbm, inv_hbm, a0_hbm, a1_hbm, y0_hbm, y1_hbm,
                 dbuf, lbuf, ibuf, vbuf, ybuf, zbuf, agg_s, sem):
    cid = lax.axis_index("c")
    sid = lax.axis_index("s")
    base = sid * EPS

    for r in range(2):
        lo = r * 5000

        def zrow(q, carry):
            for c in range(CH // 16):
                zbuf[q, pl.ds(c * 16, 16)] = jnp.zeros((16,), jnp.float32)
            return carry

        lax.fori_loop(0, 104, zrow, 0)
        for t in range(3):
            row0 = pl.multiple_of(sid * 312 + t * 104, 8)
            pltpu.sync_copy(zbuf, agg_s.at[pl.ds(row0, 104), :])

        @pl.when(sid == NS - 1)
        def _():
            pltpu.sync_copy(zbuf.at[pl.ds(0, 16), :],
                            agg_s.at[pl.ds(4992, 16), :])

        plsc.subcore_barrier()

        def step(j, carry):
            off = pl.multiple_of(base + j * KV, 8)
            pltpu.sync_copy(dst_hbm.at[pl.ds(off, KV)], dbuf)
            for q in range(KV // 16):
                qs = pl.ds(q * 16, 16)
                dvec = dbuf[qs]
                inr = (dvec >= lo) & (dvec < lo + 5000)
                lbuf[qs] = jnp.where(inr, dvec - lo, 5000)
            if r == 0:
                pltpu.async_copy(inv_hbm.at[dbuf], ibuf, sem).wait()

                @pl.when(cid == 0)
                def _():
                    pltpu.sync_copy(
                        v4_hbm.at[pl.ds(off, KV), :, pl.ds(0, CH)], vbuf)

                @pl.when(cid == 1)
                def _():
                    pltpu.sync_copy(
                        v4_hbm.at[pl.ds(off, KV), :, pl.ds(CH, CH)], vbuf)

                def edge(e, carry2):
                    wvec = ibuf[e, pl.ds(0, 16)]
                    w0 = wvec[0]
                    w1 = wvec[1]
                    w2 = wvec[2]
                    w3 = wvec[3]
                    for q2 in range(CH // 16):
                        sl = pl.ds(q2 * 16, 16)
                        acc = w0 * vbuf[e, 0, sl]
                        acc = acc + w1 * vbuf[e, 1, sl]
                        acc = acc + w2 * vbuf[e, 2, sl]
                        acc = acc + w3 * vbuf[e, 3, sl]
                        ybuf[e, sl] = acc
                    return carry2

                lax.fori_loop(0, KV, edge, 0)

                @pl.when(cid == 0)
                def _():
                    pltpu.sync_copy(ybuf, y0_hbm.at[pl.ds(off, KV), :])

                @pl.when(cid == 1)
                def _():
                    pltpu.sync_copy(ybuf, y1_hbm.at[pl.ds(off, KV), :])
            else:
                @pl.when(cid == 0)
                def _():
                    pltpu.sync_copy(y0_hbm.at[pl.ds(off, KV), :], ybuf)

                @pl.when(cid == 1)
                def _():
                    pltpu.sync_copy(y1_hbm.at[pl.ds(off, KV), :], ybuf)

            pltpu.sync_copy(ybuf, agg_s.at[lbuf], add=True)
            return carry

        lax.fori_loop(0, EPS // KV, step, 0)
        plsc.subcore_barrier()

        for t in range(3):
            loc0 = pl.multiple_of(sid * 312 + t * 104, 8)
            glb0 = pl.multiple_of(r * 5000 + sid * 312 + t * 104, 8)

            @pl.when(cid == 0)
            def _():
                pltpu.sync_copy(agg_s.at[pl.ds(loc0, 104), :],
                                a0_hbm.at[pl.ds(glb0, 104), :])

            @pl.when(cid == 1)
            def _():
                pltpu.sync_copy(agg_s.at[pl.ds(loc0, 104), :],
                                a1_hbm.at[pl.ds(glb0, 104), :])

        @pl.when(sid == NS - 1)
        def _():
            glb1 = pl.multiple_of(r * 5000 + 4992, 8)

            @pl.when(cid == 0)
            def _():
                pltpu.sync_copy(agg_s.at[pl.ds(4992, 16), :],
                                a0_hbm.at[pl.ds(glb1, 16), :])

            @pl.when(cid == 1)
            def _():
                pltpu.sync_copy(agg_s.at[pl.ds(4992, 16), :],
                                a1_hbm.at[pl.ds(glb1, 16), :])


def _sc_aggregate(v4, dst, inv2):
    kfn = pl.kernel(
        _sc_agg_body,
        out_type=[
            jax.ShapeDtypeStruct((N, CH), jnp.float32),
            jax.ShapeDtypeStruct((N, CH), jnp.float32),
            jax.ShapeDtypeStruct((E, CH), jnp.float32),
            jax.ShapeDtypeStruct((E, CH), jnp.float32),
        ],
        mesh=_mesh(),
        compiler_params=pltpu.CompilerParams(needs_layout_passes=False),
        scratch_types=[
            pltpu.VMEM((KV,), jnp.int32),
            pltpu.VMEM((KV,), jnp.int32),
            pltpu.VMEM((KV, 128), jnp.float32),
            pltpu.VMEM((KV, H, CH), jnp.float32),
            pltpu.VMEM((KV, CH), jnp.float32),
            pltpu.VMEM((104, CH), jnp.float32),
            pltpu.VMEM_SHARED((5008, CH), jnp.float32),
            pltpu.SemaphoreType.DMA,
        ],
    )
    a0, a1, _, _ = kfn(v4, dst, inv2)
    return a0, a1


def _layer(x2, src, dst, ea, wl, wr, we, attf, b, g, bb, pw):
    xl, xr = _mm(x2, wl, wr)
    gs, gd = _sc_gather(xl, xr, src, dst)
    ex4, v = _edge(gs, gd, ea, we, attf)
    dstp = jnp.pad(dst, (0, EPADL - E))
    ex4p = jnp.pad(ex4, ((0, 0), (0, EPADL - E)))
    part = _sc_denom(dstp, ex4p)
    inv = _inv(part)
    inv2 = jnp.pad(inv.reshape(N, H), ((0, 0), (0, 128 - H)))
    v4 = v.reshape(E, H, HD)
    a0, a1 = _sc_aggregate(v4, dst, inv2)
    return _post(a0, a1, b, g, bb, pw)


def kernel(x, edge_index, edge_attr, Wl1, Wr1, We1, att1, b1, bn1_g, bn1_b,
           Wl2, Wr2, We2, att2, b2, bn2_g, bn2_b, prelu_w):
    src = edge_index[0]
    dst = edge_index[1]
    r = lambda a: a.reshape(1, -1)
    h = _layer(x, src, dst, edge_attr, Wl1, Wr1, We1, r(att1), r(b1),
               r(bn1_g), r(bn1_b), None)
    h = _layer(h, src, dst, edge_attr, Wl2, Wr2, We2, r(att2), r(b2),
               r(bn2_g), r(bn2_b), r(prelu_w))
    return h


# speedup vs baseline: 6.3970x; 1.1588x over previous
import functools

import jax
import jax.numpy as jnp
from jax import lax
from jax.experimental import pallas as pl
from jax.experimental.pallas import tpu as pltpu
from jax.experimental.pallas import tpu_sc as plsc

N = 10000
E = 160000
F = 1024
H = 4
HD = 256
EDIM = 16
NC = 2
NS = 16
NW = NC * NS
EPW = E // NW
EPS = E // NS
EPAD = 5120
EPADL = NW * EPAD
KG = 8
NRING = 5
KV = 40
NACC = N * H
FH = F // 2
CH = HD // 2
BR = 312


def _mesh():
    return plsc.VectorSubcoreMesh(
        core_axis_name="c", subcore_axis_name="s", num_cores=NC, num_subcores=NS)


def _mm_body(x_ref, wl_ref, wr_ref, xl_ref, xr_ref):
    x = x_ref[...]
    xl_ref[...] = jnp.dot(x, wl_ref[...], preferred_element_type=jnp.float32)
    xr_ref[...] = jnp.dot(x, wr_ref[...], preferred_element_type=jnp.float32)


def _mm(x, wl, wr):
    n, d = x.shape
    bn = 1000
    return pl.pallas_call(
        _mm_body,
        grid=(n // bn,),
        in_specs=[
            pl.BlockSpec((bn, d), lambda i: (i, 0)),
            pl.BlockSpec((d, F), lambda i: (0, 0)),
            pl.BlockSpec((d, F), lambda i: (0, 0)),
        ],
        out_specs=[
            pl.BlockSpec((bn, F), lambda i: (i, 0)),
            pl.BlockSpec((bn, F), lambda i: (i, 0)),
        ],
        out_shape=[
            jax.ShapeDtypeStruct((n, F), jnp.float32),
            jax.ShapeDtypeStruct((n, F), jnp.float32),
        ],
    )(x, wl, wr)


def _edge_body(gs_ref, gd_ref, ea_ref, we_ref, att_ref, ex_ref, v_ref):
    gs = gs_ref[...]
    u = gs + gd_ref[...] + jnp.dot(ea_ref[...], we_ref[...],
                                   preferred_element_type=jnp.float32)
    m = jnp.maximum(u, 0.0) + 0.2 * jnp.minimum(u, 0.0)
    am = m * att_ref[...]
    for h in range(H):
        sl = slice(h * HD, (h + 1) * HD)
        lg = jnp.sum(am[:, sl], axis=1)
        exh = jnp.exp(lg)
        ex_ref[h, :] = exh
        v_ref[:, sl] = gs[:, sl] * exh[:, None]


def _edge(gs, gd, ea, we, attf):
    be = 1280
    return pl.pallas_call(
        _edge_body,
        grid=(E // be,),
        in_specs=[
            pl.BlockSpec((be, F), lambda i: (i, 0)),
            pl.BlockSpec((be, F), lambda i: (i, 0)),
            pl.BlockSpec((be, EDIM), lambda i: (i, 0)),
            pl.BlockSpec((EDIM, F), lambda i: (0, 0)),
            pl.BlockSpec((1, F), lambda i: (0, 0)),
        ],
        out_specs=[
            pl.BlockSpec((H, be), lambda i: (0, i)),
            pl.BlockSpec((be, F), lambda i: (i, 0)),
        ],
        out_shape=[
            jax.ShapeDtypeStruct((H, E), jnp.float32),
            jax.ShapeDtypeStruct((E, F), jnp.float32),
        ],
    )(gs, gd, ea, we, attf)


def _inv_body(part_ref, inv_ref):
    s = jnp.sum(part_ref[...], axis=0)
    inv_ref[...] = 1.0 / (s + 1e-16)


def _inv(part):
    return pl.pallas_call(
        _inv_body,
        out_shape=jax.ShapeDtypeStruct((NACC,), jnp.float32),
    )(part)


def _post(a0, a1, b, g, bb, pw):
    def body(*refs):
        if pw is None:
            a0_ref, a1_ref, b_ref, g_ref, bb_ref, out_ref = refs
        else:
            a0_ref, a1_ref, b_ref, g_ref, bb_ref, pw_ref, out_ref = refs
        h = (jnp.concatenate([a0_ref[...], a1_ref[...]], axis=1) * 0.25
             + b_ref[...])
        mu = jnp.mean(h, axis=0, keepdims=True)
        d = h - mu
        var = jnp.mean(d * d, axis=0, keepdims=True)
        hn = d * lax.rsqrt(var + 1e-5) * g_ref[...] + bb_ref[...]
        if pw is None:
            out_ref[...] = hn
        else:
            out_ref[...] = (jnp.maximum(hn, 0.0)
                            + pw_ref[...] * jnp.minimum(hn, 0.0))

    args = (a0, a1, b, g, bb) if pw is None else (a0, a1, b, g, bb, pw)
    return pl.pallas_call(
        body,
        out_shape=jax.ShapeDtypeStruct((N, HD), jnp.float32),
    )(*args)


def _sc_gather_body(xl_hbm, xr_hbm, src_hbm, dst_hbm, gs_hbm, gd_hbm,
                    *scr):
    wid = lax.axis_index("s") * NC + lax.axis_index("c")
    base = wid * EPW
    bufs = tuple(scr[5 * s:5 * s + 5] for s in range(NRING))

    def step(k, carry):
        gcp = []
        for s in range(NRING):
            sidx, didx, rl, rr, gsem = bufs[s]
            off = pl.multiple_of(base + (k * NRING + s) * KG, 8)
            pltpu.sync_copy(src_hbm.at[pl.ds(off, KG)], sidx)
            pltpu.sync_copy(dst_hbm.at[pl.ds(off, KG)], didx)
            c0 = pltpu.async_copy(xl_hbm.at[sidx], rl, gsem)
            c1 = pltpu.async_copy(xr_hbm.at[didx], rr, gsem)
            gcp.append((c0, c1, off))
        wcp = []
        for s in range(NRING):
            rl, rr, wsem = bufs[s][2], bufs[s][3], bufs[s][4]

            c0, c1, off = gcp[s]
            c0.wait()
            c1.wait()
            w0 = pltpu.async_copy(rl, gs_hbm.at[pl.ds(off, KG), :], wsem)
            w1 = pltpu.async_copy(rr, gd_hbm.at[pl.ds(off, KG), :], wsem)
            wcp.append((w0, w1))
        for w0, w1 in wcp:
            w0.wait()
            w1.wait()
        return carry

    lax.fori_loop(0, EPW // KG // NRING, step, 0)

def _sc_gather(xl, xr, src, dst):
    ring = []
    for _ in range(NRING):
        ring += [
            pltpu.VMEM((KG,), jnp.int32),
            pltpu.VMEM((KG,), jnp.int32),
            pltpu.VMEM((KG, F), jnp.float32),
            pltpu.VMEM((KG, F), jnp.float32),
            pltpu.SemaphoreType.DMA,
        ]
    kfn = pl.kernel(
        _sc_gather_body,
        out_type=[
            jax.ShapeDtypeStruct((E, F), jnp.float32),
            jax.ShapeDtypeStruct((E, F), jnp.float32),
        ],
        mesh=_mesh(),
        compiler_params=pltpu.CompilerParams(needs_layout_passes=False),
        scratch_types=ring,
    )
    return kfn(xl, xr, src, dst)


def _sc_denom_body(dst_hbm, ex_hbm, part_hbm, acc, dbuf, ebuf):
    wid = lax.axis_index("s") * NC + lax.axis_index("c")

    def zero(k, carry):
        acc[pl.ds(pl.multiple_of(k * 16, 8), 16)] = jnp.zeros((16,), jnp.float32)
        return carry

    lax.fori_loop(0, NACC // 16, zero, 0)

    off0 = pl.multiple_of(wid * EPAD, 128)
    pltpu.sync_copy(dst_hbm.at[pl.ds(off0, EPAD)], dbuf)
    pltpu.sync_copy(ex_hbm.at[:, pl.ds(off0, EPAD)], ebuf)

    def step(i, carry):
        off = pl.multiple_of(i * 16, 8)
        dvec = dbuf[pl.ds(off, 16)]
        fours = dvec * H
        for h in range(H):
            plsc.addupdate_scatter(acc, [fours + h], ebuf[h, pl.ds(off, 16)])
        return carry

    lax.fori_loop(0, EPAD // 16, step, 0)
    pltpu.sync_copy(acc, part_hbm.at[wid])


def _sc_denom(dstp, ex4p):
    kfn = pl.kernel(
        _sc_denom_body,
        out_type=jax.ShapeDtypeStruct((NW, NACC), jnp.float32),
        mesh=_mesh(),
        compiler_params=pltpu.CompilerParams(needs_layout_passes=False),
        scratch_types=[
            pltpu.VMEM((NACC,), jnp.float32),
            pltpu.VMEM((EPAD,), jnp.int32),
            pltpu.VMEM((H, EPAD), jnp.float32),
        ],
    )
    return kfn(dstp, ex4p)


def _sc_agg_body(v4_hbm, dst_hbm, inv_hbm, a0_hbm, a1_hbm, y0_hbm, y1_hbm,
                 dbuf0, lbuf0, ibuf0, va0, vb0, vc0, vd0, ybuf0,
                 dbuf1, lbuf1, ibuf1, va1, vb1, vc1, vd1, ybuf1,
                 zbuf, agg_s, sem0, sem1):
    cid = lax.axis_index("c")
    sid = lax.axis_index("s")
    base = sid * EPS
    bufs = ((dbuf0, lbuf0, ibuf0, (va0, vb0, vc0, vd0), ybuf0, sem0),
            (dbuf1, lbuf1, ibuf1, (va1, vb1, vc1, vd1), ybuf1, sem1))

    def local_idx(lo, b):
        dbuf, lbuf = bufs[b][0], bufs[b][1]
        for q0 in (0, 16, 24):
            qs = pl.ds(q0, 16)
            dvec = dbuf[qs]
            inr = (dvec >= lo) & (dvec < lo + 5000)
            lbuf[qs] = jnp.where(inr, dvec - lo, 5000)

    def compute_y(b):
        ibuf, vbufs, ybuf = bufs[b][2], bufs[b][3], bufs[b][4]

        def edge(e, carry2):
            wvec = ibuf[e, pl.ds(0, 16)]
            w0 = wvec[0]
            w1 = wvec[1]
            w2 = wvec[2]
            w3 = wvec[3]
            for q2 in range(CH // 16):
                sl = pl.ds(q2 * 16, 16)
                acc = w0 * vbufs[0][e, sl]
                acc = acc + w1 * vbufs[1][e, sl]
                acc = acc + w2 * vbufs[2][e, sl]
                acc = acc + w3 * vbufs[3][e, sl]
                ybuf[e, sl] = acc
            return carry2

        lax.fori_loop(0, KV, edge, 0)

    for r in range(2):
        lo = r * 5000

        def zrow(q, carry):
            for c in range(CH // 16):
                zbuf[q, pl.ds(c * 16, 16)] = jnp.zeros((16,), jnp.float32)
            return carry

        lax.fori_loop(0, 24, zrow, 0)
        for t in range(13):
            row0 = pl.multiple_of(sid * 312 + t * 24, 8)
            pltpu.sync_copy(zbuf, agg_s.at[pl.ds(row0, 24), :])

        @pl.when(sid == NS - 1)
        def _():
            pltpu.sync_copy(zbuf.at[pl.ds(0, 16), :],
                            agg_s.at[pl.ds(4992, 16), :])

        plsc.subcore_barrier()

        if r == 0:
            def load0(off, b):
                dbuf, ibuf, vbufs, sem = bufs[b][0], bufs[b][2], bufs[b][3], bufs[b][5]
                pltpu.sync_copy(dst_hbm.at[pl.ds(off, KV)], dbuf)
                pltpu.async_copy(inv_hbm.at[dbuf], ibuf, sem).wait()

                @pl.when(cid == 0)
                def _():
                    for h in range(H):
                        pltpu.sync_copy(
                            v4_hbm.at[pl.ds(off, KV), pl.ds(h * HD, CH)],
                            vbufs[h])

                @pl.when(cid == 1)
                def _():
                    for h in range(H):
                        pltpu.sync_copy(
                            v4_hbm.at[pl.ds(off, KV), pl.ds(h * HD + CH, CH)],
                            vbufs[h])

                return None

            def fin0(off, b, cps):
                ybuf = bufs[b][4]
                local_idx(lo, b)
                compute_y(b)

                @pl.when(cid == 0)
                def _():
                    pltpu.sync_copy(ybuf, y0_hbm.at[pl.ds(off, KV), :])

                @pl.when(cid == 1)
                def _():
                    pltpu.sync_copy(ybuf, y1_hbm.at[pl.ds(off, KV), :])

                pltpu.sync_copy(ybuf, agg_s.at[bufs[b][1]], add=True)

            def step0(k, carry):
                offa = pl.multiple_of(base + (2 * k) * KV, 8)
                offb = pl.multiple_of(base + (2 * k + 1) * KV, 8)
                ca = load0(offa, 0)
                cb = load0(offb, 1)
                fin0(offa, 0, ca)
                fin0(offb, 1, cb)
                return carry

            lax.fori_loop(0, EPS // KV // 2, step0, 0)
        else:
            def load1(off, b):
                dbuf, _, _, _, ybuf, sem = bufs[b]
                pltpu.sync_copy(dst_hbm.at[pl.ds(off, KV)], dbuf)

                @pl.when(cid == 0)
                def _():
                    pltpu.async_copy(y0_hbm.at[pl.ds(off, KV), :], ybuf, sem)

                @pl.when(cid == 1)
                def _():
                    pltpu.async_copy(y1_hbm.at[pl.ds(off, KV), :], ybuf, sem)

                cy = pltpu.make_async_copy(y0_hbm.at[pl.ds(off, KV), :], ybuf,
                                           sem)
                return cy

            def fin1(off, b, cy):
                ybuf = bufs[b][4]
                cy.wait()
                local_idx(lo, b)
                pltpu.sync_copy(ybuf, agg_s.at[bufs[b][1]], add=True)

            def step1(k, carry):
                offa = pl.multiple_of(base + (2 * k) * KV, 8)
                offb = pl.multiple_of(base + (2 * k + 1) * KV, 8)
                ca = load1(offa, 0)
                cb = load1(offb, 1)
                fin1(offa, 0, ca)
                fin1(offb, 1, cb)
                return carry

            lax.fori_loop(0, EPS // KV // 2, step1, 0)

        plsc.subcore_barrier()

        for t in range(3):
            loc0 = pl.multiple_of(sid * 312 + t * 104, 8)
            glb0 = pl.multiple_of(r * 5000 + sid * 312 + t * 104, 8)

            @pl.when(cid == 0)
            def _():
                pltpu.sync_copy(agg_s.at[pl.ds(loc0, 104), :],
                                a0_hbm.at[pl.ds(glb0, 104), :])

            @pl.when(cid == 1)
            def _():
                pltpu.sync_copy(agg_s.at[pl.ds(loc0, 104), :],
                                a1_hbm.at[pl.ds(glb0, 104), :])

        @pl.when(sid == NS - 1)
        def _():
            glb1 = pl.multiple_of(r * 5000 + 4992, 8)

            @pl.when(cid == 0)
            def _():
                pltpu.sync_copy(agg_s.at[pl.ds(4992, 16), :],
                                a0_hbm.at[pl.ds(glb1, 16), :])

            @pl.when(cid == 1)
            def _():
                pltpu.sync_copy(agg_s.at[pl.ds(4992, 16), :],
                                a1_hbm.at[pl.ds(glb1, 16), :])


def _sc_aggregate(v4, dst, inv2):
    dbl = lambda: [
        pltpu.VMEM((KV,), jnp.int32),
        pltpu.VMEM((KV,), jnp.int32),
        pltpu.VMEM((KV, 128), jnp.float32),
        pltpu.VMEM((KV, CH), jnp.float32),
        pltpu.VMEM((KV, CH), jnp.float32),
        pltpu.VMEM((KV, CH), jnp.float32),
        pltpu.VMEM((KV, CH), jnp.float32),
        pltpu.VMEM((KV, CH), jnp.float32),
    ]
    kfn = pl.kernel(
        _sc_agg_body,
        out_type=[
            jax.ShapeDtypeStruct((N, CH), jnp.float32),
            jax.ShapeDtypeStruct((N, CH), jnp.float32),
            jax.ShapeDtypeStruct((E, CH), jnp.float32),
            jax.ShapeDtypeStruct((E, CH), jnp.float32),
        ],
        mesh=_mesh(),
        compiler_params=pltpu.CompilerParams(needs_layout_passes=False),
        scratch_types=dbl() + dbl() + [
            pltpu.VMEM((24, CH), jnp.float32),
            pltpu.VMEM_SHARED((5008, CH), jnp.float32),
            pltpu.SemaphoreType.DMA,
            pltpu.SemaphoreType.DMA,
        ],
    )
    a0, a1, _, _ = kfn(v4, dst, inv2)
    return a0, a1


def _layer(x2, src, dst, ea, wl, wr, we, attf, b, g, bb, pw):
    xl, xr = _mm(x2, wl, wr)
    gs, gd = _sc_gather(xl, xr, src, dst)
    ex4, v = _edge(gs, gd, ea, we, attf)
    dstp = jnp.pad(dst, (0, EPADL - E))
    ex4p = jnp.pad(ex4, ((0, 0), (0, EPADL - E)))
    part = _sc_denom(dstp, ex4p)
    inv = _inv(part)
    inv2 = jnp.pad(inv.reshape(N, H), ((0, 0), (0, 128 - H)))
    a0, a1 = _sc_aggregate(v, dst, inv2)
    return _post(a0, a1, b, g, bb, pw)


def kernel(x, edge_index, edge_attr, Wl1, Wr1, We1, att1, b1, bn1_g, bn1_b,
           Wl2, Wr2, We2, att2, b2, bn2_g, bn2_b, prelu_w):
    src = edge_index[0]
    dst = edge_index[1]
    r = lambda a: a.reshape(1, -1)
    h = _layer(x, src, dst, edge_attr, Wl1, Wr1, We1, r(att1), r(b1),
               r(bn1_g), r(bn1_b), None)
    h = _layer(h, src, dst, edge_attr, Wl2, Wr2, We2, r(att2), r(b2),
               r(bn2_g), r(bn2_b), r(prelu_w))
    return h


# speedup vs baseline: 8.7053x; 1.3608x over previous
import functools

import jax
import jax.numpy as jnp
from jax import lax
from jax.experimental import pallas as pl
from jax.experimental.pallas import tpu as pltpu
from jax.experimental.pallas import tpu_sc as plsc

N = 10000
E = 160000
F = 1024
H = 4
HD = 256
EDIM = 16
NC = 2
NS = 16
NW = NC * NS
EPW = E // NW
EPS = E // NS
EPAD = 5120
EPADL = NW * EPAD
KG = 8
NRING = 5
KV = 40
NACC = N * H
FH = F // 2
CH = HD // 2
BR = 312


def _mesh():
    return plsc.VectorSubcoreMesh(
        core_axis_name="c", subcore_axis_name="s", num_cores=NC, num_subcores=NS)


def _mm_body(x_ref, wl_ref, wr_ref, xl_ref, xr_ref):
    x = x_ref[...]
    xl_ref[...] = jnp.dot(x, wl_ref[...], preferred_element_type=jnp.float32)
    xr_ref[...] = jnp.dot(x, wr_ref[...], preferred_element_type=jnp.float32)


def _mm(x, wl, wr):
    n, d = x.shape
    bn = 1000
    return pl.pallas_call(
        _mm_body,
        grid=(n // bn,),
        in_specs=[
            pl.BlockSpec((bn, d), lambda i: (i, 0)),
            pl.BlockSpec((d, F), lambda i: (0, 0)),
            pl.BlockSpec((d, F), lambda i: (0, 0)),
        ],
        out_specs=[
            pl.BlockSpec((bn, F), lambda i: (i, 0)),
            pl.BlockSpec((bn, F), lambda i: (i, 0)),
        ],
        out_shape=[
            jax.ShapeDtypeStruct((n, F), jnp.float32),
            jax.ShapeDtypeStruct((n, F), jnp.float32),
        ],
    )(x, wl, wr)


def _edge_body(gs_ref, gd_ref, ea_ref, we_ref, att_ref, ex_ref, v_ref):
    gs = gs_ref[...]
    u = gs + gd_ref[...] + jnp.dot(ea_ref[...], we_ref[...],
                                   preferred_element_type=jnp.float32)
    m = jnp.maximum(u, 0.0) + 0.2 * jnp.minimum(u, 0.0)
    am = m * att_ref[...]
    for h in range(H):
        sl = slice(h * HD, (h + 1) * HD)
        lg = jnp.sum(am[:, sl], axis=1)
        exh = jnp.exp(lg)
        ex_ref[h, :] = exh
        v_ref[:, sl] = gs[:, sl] * exh[:, None]


def _edge(gs, gd, ea, we, attf):
    be = 1280
    return pl.pallas_call(
        _edge_body,
        grid=(E // be,),
        in_specs=[
            pl.BlockSpec((be, F), lambda i: (i, 0)),
            pl.BlockSpec((be, F), lambda i: (i, 0)),
            pl.BlockSpec((be, EDIM), lambda i: (i, 0)),
            pl.BlockSpec((EDIM, F), lambda i: (0, 0)),
            pl.BlockSpec((1, F), lambda i: (0, 0)),
        ],
        out_specs=[
            pl.BlockSpec((H, be), lambda i: (0, i)),
            pl.BlockSpec((be, F), lambda i: (i, 0)),
        ],
        out_shape=[
            jax.ShapeDtypeStruct((H, E), jnp.float32),
            jax.ShapeDtypeStruct((E, F), jnp.float32),
        ],
    )(gs, gd, ea, we, attf)


def _inv_body(part_ref, inv_ref):
    s = jnp.sum(part_ref[...], axis=0)
    inv_ref[...] = 1.0 / (s + 1e-16)


def _inv(part):
    return pl.pallas_call(
        _inv_body,
        out_shape=jax.ShapeDtypeStruct((NACC,), jnp.float32),
    )(part)


def _post(a0, a1, b, g, bb, pw):
    def body(*refs):
        if pw is None:
            a0_ref, a1_ref, b_ref, g_ref, bb_ref, out_ref = refs
        else:
            a0_ref, a1_ref, b_ref, g_ref, bb_ref, pw_ref, out_ref = refs
        h = (jnp.concatenate([a0_ref[...], a1_ref[...]], axis=1) * 0.25
             + b_ref[...])
        mu = jnp.mean(h, axis=0, keepdims=True)
        d = h - mu
        var = jnp.mean(d * d, axis=0, keepdims=True)
        hn = d * lax.rsqrt(var + 1e-5) * g_ref[...] + bb_ref[...]
        if pw is None:
            out_ref[...] = hn
        else:
            out_ref[...] = (jnp.maximum(hn, 0.0)
                            + pw_ref[...] * jnp.minimum(hn, 0.0))

    args = (a0, a1, b, g, bb) if pw is None else (a0, a1, b, g, bb, pw)
    return pl.pallas_call(
        body,
        out_shape=jax.ShapeDtypeStruct((N, HD), jnp.float32),
    )(*args)


def _sc_gather_body(xl_hbm, xr_hbm, src_hbm, dst_hbm, gs_hbm, gd_hbm,
                    *scr):
    wid = lax.axis_index("s") * NC + lax.axis_index("c")
    base = wid * EPW
    sbuf, dbuf = scr[0], scr[1]
    bufs = tuple(scr[2 + 3 * s:2 + 3 * s + 3] for s in range(NRING))

    off0 = pl.multiple_of(base, 8)
    pltpu.sync_copy(src_hbm.at[pl.ds(off0, EPW)], sbuf)
    pltpu.sync_copy(dst_hbm.at[pl.ds(off0, EPW)], dbuf)

    def step(k, carry):
        gcp = []
        for s in range(NRING):
            rl, rr, sem = bufs[s]
            loc = pl.multiple_of((k * NRING + s) * KG, 8)
            off = pl.multiple_of(base + (k * NRING + s) * KG, 8)
            c0 = pltpu.async_copy(xl_hbm.at[sbuf.at[pl.ds(loc, KG)]], rl, sem)
            c1 = pltpu.async_copy(xr_hbm.at[dbuf.at[pl.ds(loc, KG)]], rr, sem)
            gcp.append((c0, c1, off))
        wcp = []
        for s in range(NRING):
            rl, rr, sem = bufs[s]
            c0, c1, off = gcp[s]
            c0.wait()
            c1.wait()
            w0 = pltpu.async_copy(rl, gs_hbm.at[pl.ds(off, KG), :], sem)
            w1 = pltpu.async_copy(rr, gd_hbm.at[pl.ds(off, KG), :], sem)
            wcp.append((w0, w1))
        for w0, w1 in wcp:
            w0.wait()
            w1.wait()
        return carry

    lax.fori_loop(0, EPW // KG // NRING, step, 0)


def _sc_gather(xl, xr, src, dst):
    ring = [
        pltpu.VMEM((EPW,), jnp.int32),
        pltpu.VMEM((EPW,), jnp.int32),
    ]
    for _ in range(NRING):
        ring += [
            pltpu.VMEM((KG, F), jnp.float32),
            pltpu.VMEM((KG, F), jnp.float32),
            pltpu.SemaphoreType.DMA,
        ]
    kfn = pl.kernel(
        _sc_gather_body,
        out_type=[
            jax.ShapeDtypeStruct((E, F), jnp.float32),
            jax.ShapeDtypeStruct((E, F), jnp.float32),
        ],
        mesh=_mesh(),
        compiler_params=pltpu.CompilerParams(needs_layout_passes=False),
        scratch_types=ring,
    )
    return kfn(xl, xr, src, dst)


def _sc_denom_body(dst_hbm, ex_hbm, part_hbm, acc, dbuf, ebuf):
    wid = lax.axis_index("s") * NC + lax.axis_index("c")

    def zero(k, carry):
        acc[pl.ds(pl.multiple_of(k * 16, 8), 16)] = jnp.zeros((16,), jnp.float32)
        return carry

    lax.fori_loop(0, NACC // 16, zero, 0)

    off0 = pl.multiple_of(wid * EPAD, 128)
    pltpu.sync_copy(dst_hbm.at[pl.ds(off0, EPAD)], dbuf)
    pltpu.sync_copy(ex_hbm.at[:, pl.ds(off0, EPAD)], ebuf)

    def step(i, carry):
        off = pl.multiple_of(i * 16, 8)
        dvec = dbuf[pl.ds(off, 16)]
        fours = dvec * H
        for h in range(H):
            plsc.addupdate_scatter(acc, [fours + h], ebuf[h, pl.ds(off, 16)])
        return carry

    lax.fori_loop(0, EPAD // 16, step, 0)
    pltpu.sync_copy(acc, part_hbm.at[wid])


def _sc_denom(dstp, ex4p):
    kfn = pl.kernel(
        _sc_denom_body,
        out_type=jax.ShapeDtypeStruct((NW, NACC), jnp.float32),
        mesh=_mesh(),
        compiler_params=pltpu.CompilerParams(needs_layout_passes=False),
        scratch_types=[
            pltpu.VMEM((NACC,), jnp.float32),
            pltpu.VMEM((EPAD,), jnp.int32),
            pltpu.VMEM((H, EPAD), jnp.float32),
        ],
    )
    return kfn(dstp, ex4p)


def _sc_agg_body(v4_hbm, dst_hbm, inv_hbm, a0_hbm, a1_hbm, y0_hbm, y1_hbm,
                 dbuf0, lbuf0, ibuf0, va0, vb0, vc0, vd0, ybuf0,
                 dbuf1, lbuf1, ibuf1, va1, vb1, vc1, vd1, ybuf1,
                 zbuf, agg_s, sem0, sem1):
    cid = lax.axis_index("c")
    sid = lax.axis_index("s")
    base = sid * EPS
    bufs = ((dbuf0, lbuf0, ibuf0, (va0, vb0, vc0, vd0), ybuf0, sem0),
            (dbuf1, lbuf1, ibuf1, (va1, vb1, vc1, vd1), ybuf1, sem1))

    def local_idx(lo, b):
        dbuf, lbuf = bufs[b][0], bufs[b][1]
        for q0 in (0, 16, 24):
            qs = pl.ds(q0, 16)
            dvec = dbuf[qs]
            inr = (dvec >= lo) & (dvec < lo + 5000)
            lbuf[qs] = jnp.where(inr, dvec - lo, 5000)

    def compute_y(b):
        ibuf, vbufs, ybuf = bufs[b][2], bufs[b][3], bufs[b][4]

        def edge(e, carry2):
            wvec = ibuf[e, pl.ds(0, 16)]
            w0 = wvec[0]
            w1 = wvec[1]
            w2 = wvec[2]
            w3 = wvec[3]
            for q2 in range(CH // 16):
                sl = pl.ds(q2 * 16, 16)
                acc = w0 * vbufs[0][e, sl]
                acc = acc + w1 * vbufs[1][e, sl]
                acc = acc + w2 * vbufs[2][e, sl]
                acc = acc + w3 * vbufs[3][e, sl]
                ybuf[e, sl] = acc
            return carry2

        lax.fori_loop(0, KV, edge, 0)

    for r in range(2):
        lo = r * 5000

        def zrow(q, carry):
            for c in range(CH // 16):
                zbuf[q, pl.ds(c * 16, 16)] = jnp.zeros((16,), jnp.float32)
            return carry

        lax.fori_loop(0, 24, zrow, 0)
        for t in range(13):
            row0 = pl.multiple_of(sid * 312 + t * 24, 8)
            pltpu.sync_copy(zbuf, agg_s.at[pl.ds(row0, 24), :])

        @pl.when(sid == NS - 1)
        def _():
            pltpu.sync_copy(zbuf.at[pl.ds(0, 16), :],
                            agg_s.at[pl.ds(4992, 16), :])

        plsc.subcore_barrier()

        if r == 0:
            def load0(off, b):
                dbuf, ibuf, vbufs, sem = bufs[b][0], bufs[b][2], bufs[b][3], bufs[b][5]
                pltpu.sync_copy(dst_hbm.at[pl.ds(off, KV)], dbuf)
                ci = pltpu.async_copy(inv_hbm.at[dbuf], ibuf, sem)
                cvs = []

                @pl.when(cid == 0)
                def _():
                    for h in range(H):
                        pltpu.async_copy(
                            v4_hbm.at[pl.ds(off, KV), pl.ds(h * HD, CH)],
                            vbufs[h], sem)

                @pl.when(cid == 1)
                def _():
                    for h in range(H):
                        pltpu.async_copy(
                            v4_hbm.at[pl.ds(off, KV), pl.ds(h * HD + CH, CH)],
                            vbufs[h], sem)

                for h in range(H):
                    cvs.append(pltpu.make_async_copy(
                        v4_hbm.at[pl.ds(off, KV), pl.ds(h * HD, CH)],
                        vbufs[h], sem))
                return [ci] + cvs

            def fin0(off, b, cps):
                ybuf = bufs[b][4]
                for c in cps:
                    c.wait()
                local_idx(lo, b)
                compute_y(b)

                @pl.when(cid == 0)
                def _():
                    pltpu.sync_copy(ybuf, y0_hbm.at[pl.ds(off, KV), :])

                @pl.when(cid == 1)
                def _():
                    pltpu.sync_copy(ybuf, y1_hbm.at[pl.ds(off, KV), :])

                pltpu.sync_copy(ybuf, agg_s.at[bufs[b][1]], add=True)

            def step0(k, carry):
                offa = pl.multiple_of(base + (2 * k) * KV, 8)
                offb = pl.multiple_of(base + (2 * k + 1) * KV, 8)
                ca = load0(offa, 0)
                cb = load0(offb, 1)
                fin0(offa, 0, ca)
                fin0(offb, 1, cb)
                return carry

            lax.fori_loop(0, EPS // KV // 2, step0, 0)
        else:
            def load1(off, b):
                dbuf, _, _, _, ybuf, sem = bufs[b]
                pltpu.sync_copy(dst_hbm.at[pl.ds(off, KV)], dbuf)

                @pl.when(cid == 0)
                def _():
                    pltpu.async_copy(y0_hbm.at[pl.ds(off, KV), :], ybuf, sem)

                @pl.when(cid == 1)
                def _():
                    pltpu.async_copy(y1_hbm.at[pl.ds(off, KV), :], ybuf, sem)

                cy = pltpu.make_async_copy(y0_hbm.at[pl.ds(off, KV), :], ybuf,
                                           sem)
                return cy

            def fin1(off, b, cy):
                ybuf = bufs[b][4]
                cy.wait()
                local_idx(lo, b)
                pltpu.sync_copy(ybuf, agg_s.at[bufs[b][1]], add=True)

            def step1(k, carry):
                offa = pl.multiple_of(base + (2 * k) * KV, 8)
                offb = pl.multiple_of(base + (2 * k + 1) * KV, 8)
                ca = load1(offa, 0)
                cb = load1(offb, 1)
                fin1(offa, 0, ca)
                fin1(offb, 1, cb)
                return carry

            lax.fori_loop(0, EPS // KV // 2, step1, 0)

        plsc.subcore_barrier()

        for t in range(3):
            loc0 = pl.multiple_of(sid * 312 + t * 104, 8)
            glb0 = pl.multiple_of(r * 5000 + sid * 312 + t * 104, 8)

            @pl.when(cid == 0)
            def _():
                pltpu.sync_copy(agg_s.at[pl.ds(loc0, 104), :],
                                a0_hbm.at[pl.ds(glb0, 104), :])

            @pl.when(cid == 1)
            def _():
                pltpu.sync_copy(agg_s.at[pl.ds(loc0, 104), :],
                                a1_hbm.at[pl.ds(glb0, 104), :])

        @pl.when(sid == NS - 1)
        def _():
            glb1 = pl.multiple_of(r * 5000 + 4992, 8)

            @pl.when(cid == 0)
            def _():
                pltpu.sync_copy(agg_s.at[pl.ds(4992, 16), :],
                                a0_hbm.at[pl.ds(glb1, 16), :])

            @pl.when(cid == 1)
            def _():
                pltpu.sync_copy(agg_s.at[pl.ds(4992, 16), :],
                                a1_hbm.at[pl.ds(glb1, 16), :])


def _sc_aggregate(v4, dst, inv2):
    dbl = lambda: [
        pltpu.VMEM((KV,), jnp.int32),
        pltpu.VMEM((KV,), jnp.int32),
        pltpu.VMEM((KV, 128), jnp.float32),
        pltpu.VMEM((KV, CH), jnp.float32),
        pltpu.VMEM((KV, CH), jnp.float32),
        pltpu.VMEM((KV, CH), jnp.float32),
        pltpu.VMEM((KV, CH), jnp.float32),
        pltpu.VMEM((KV, CH), jnp.float32),
    ]
    kfn = pl.kernel(
        _sc_agg_body,
        out_type=[
            jax.ShapeDtypeStruct((N, CH), jnp.float32),
            jax.ShapeDtypeStruct((N, CH), jnp.float32),
            jax.ShapeDtypeStruct((E, CH), jnp.float32),
            jax.ShapeDtypeStruct((E, CH), jnp.float32),
        ],
        mesh=_mesh(),
        compiler_params=pltpu.CompilerParams(needs_layout_passes=False),
        scratch_types=dbl() + dbl() + [
            pltpu.VMEM((24, CH), jnp.float32),
            pltpu.VMEM_SHARED((5008, CH), jnp.float32),
            pltpu.SemaphoreType.DMA,
            pltpu.SemaphoreType.DMA,
        ],
    )
    a0, a1, _, _ = kfn(v4, dst, inv2)
    return a0, a1


def _layer(x2, src, dst, ea, wl, wr, we, attf, b, g, bb, pw):
    xl, xr = _mm(x2, wl, wr)
    gs, gd = _sc_gather(xl, xr, src, dst)
    ex4, v = _edge(gs, gd, ea, we, attf)
    dstp = jnp.pad(dst, (0, EPADL - E))
    ex4p = jnp.pad(ex4, ((0, 0), (0, EPADL - E)))
    part = _sc_denom(dstp, ex4p)
    inv = _inv(part)
    inv2 = jnp.pad(inv.reshape(N, H), ((0, 0), (0, 128 - H)))
    a0, a1 = _sc_aggregate(v, dst, inv2)
    return _post(a0, a1, b, g, bb, pw)


def kernel(x, edge_index, edge_attr, Wl1, Wr1, We1, att1, b1, bn1_g, bn1_b,
           Wl2, Wr2, We2, att2, b2, bn2_g, bn2_b, prelu_w):
    src = edge_index[0]
    dst = edge_index[1]
    r = lambda a: a.reshape(1, -1)
    h = _layer(x, src, dst, edge_attr, Wl1, Wr1, We1, r(att1), r(b1),
               r(bn1_g), r(bn1_b), None)
    h = _layer(h, src, dst, edge_attr, Wl2, Wr2, We2, r(att2), r(b2),
               r(bn2_g), r(bn2_b), r(prelu_w))
    return h


# speedup vs baseline: 8.7055x; 1.0000x over previous
import functools

import jax
import jax.numpy as jnp
from jax import lax
from jax.experimental import pallas as pl
from jax.experimental.pallas import tpu as pltpu
from jax.experimental.pallas import tpu_sc as plsc

N = 10000
E = 160000
F = 1024
H = 4
HD = 256
EDIM = 16
NC = 2
NS = 16
NW = NC * NS
EPW = E // NW
EPS = E // NS
EPAD = 5120
EPADL = NW * EPAD
KG = 8
NRING = 5
KV = 40
NACC = N * H
FH = F // 2
CH = HD // 2
BR = 312


def _mesh():
    return plsc.VectorSubcoreMesh(
        core_axis_name="c", subcore_axis_name="s", num_cores=NC, num_subcores=NS)


def _mm_body(x_ref, wl_ref, wr_ref, xl_ref, xr_ref):
    x = x_ref[...]
    xl_ref[...] = jnp.dot(x, wl_ref[...], preferred_element_type=jnp.float32)
    xr_ref[...] = jnp.dot(x, wr_ref[...], preferred_element_type=jnp.float32)


def _mm(x, wl, wr):
    n, d = x.shape
    bn = 1000
    return pl.pallas_call(
        _mm_body,
        grid=(n // bn,),
        in_specs=[
            pl.BlockSpec((bn, d), lambda i: (i, 0)),
            pl.BlockSpec((d, F), lambda i: (0, 0)),
            pl.BlockSpec((d, F), lambda i: (0, 0)),
        ],
        out_specs=[
            pl.BlockSpec((bn, F), lambda i: (i, 0)),
            pl.BlockSpec((bn, F), lambda i: (i, 0)),
        ],
        out_shape=[
            jax.ShapeDtypeStruct((n, F), jnp.float32),
            jax.ShapeDtypeStruct((n, F), jnp.float32),
        ],
    )(x, wl, wr)


def _edge_body(gs_ref, gd_ref, ea_ref, we_ref, att_ref, ex_ref, v_ref):
    gs = gs_ref[...]
    u = gs + gd_ref[...] + jnp.dot(ea_ref[...], we_ref[...],
                                   preferred_element_type=jnp.float32)
    m = jnp.maximum(u, 0.0) + 0.2 * jnp.minimum(u, 0.0)
    am = m * att_ref[...]
    for h in range(H):
        sl = slice(h * HD, (h + 1) * HD)
        lg = jnp.sum(am[:, sl], axis=1)
        exh = jnp.exp(lg)
        ex_ref[h, :] = exh
        v_ref[:, sl] = gs[:, sl] * exh[:, None]


def _edge(gs, gd, ea, we, attf):
    be = 1280
    return pl.pallas_call(
        _edge_body,
        grid=(E // be,),
        in_specs=[
            pl.BlockSpec((be, F), lambda i: (i, 0)),
            pl.BlockSpec((be, F), lambda i: (i, 0)),
            pl.BlockSpec((be, EDIM), lambda i: (i, 0)),
            pl.BlockSpec((EDIM, F), lambda i: (0, 0)),
            pl.BlockSpec((1, F), lambda i: (0, 0)),
        ],
        out_specs=[
            pl.BlockSpec((H, be), lambda i: (0, i)),
            pl.BlockSpec((be, F), lambda i: (i, 0)),
        ],
        out_shape=[
            jax.ShapeDtypeStruct((H, E), jnp.float32),
            jax.ShapeDtypeStruct((E, F), jnp.float32),
        ],
    )(gs, gd, ea, we, attf)


def _inv_body(part_ref, inv_ref):
    s = jnp.sum(part_ref[...], axis=0)
    inv_ref[...] = 1.0 / (s + 1e-16)


def _inv(part):
    return pl.pallas_call(
        _inv_body,
        out_shape=jax.ShapeDtypeStruct((NACC,), jnp.float32),
    )(part)


def _post(a0, a1, b, g, bb, pw):
    def body(*refs):
        if pw is None:
            a0_ref, a1_ref, b_ref, g_ref, bb_ref, out_ref = refs
        else:
            a0_ref, a1_ref, b_ref, g_ref, bb_ref, pw_ref, out_ref = refs
        h = (jnp.concatenate([a0_ref[...], a1_ref[...]], axis=1) * 0.25
             + b_ref[...])
        mu = jnp.mean(h, axis=0, keepdims=True)
        d = h - mu
        var = jnp.mean(d * d, axis=0, keepdims=True)
        hn = d * lax.rsqrt(var + 1e-5) * g_ref[...] + bb_ref[...]
        if pw is None:
            out_ref[...] = hn
        else:
            out_ref[...] = (jnp.maximum(hn, 0.0)
                            + pw_ref[...] * jnp.minimum(hn, 0.0))

    args = (a0, a1, b, g, bb) if pw is None else (a0, a1, b, g, bb, pw)
    return pl.pallas_call(
        body,
        out_shape=jax.ShapeDtypeStruct((N, HD), jnp.float32),
    )(*args)


def _sc_gather_body(xl_hbm, xr_hbm, src_hbm, dst_hbm, gs_hbm, gd_hbm,
                    *scr):
    wid = lax.axis_index("s") * NC + lax.axis_index("c")
    base = wid * EPW
    sbuf, dbuf = scr[0], scr[1]
    bufs = tuple(scr[2 + 3 * s:2 + 3 * s + 3] for s in range(NRING))

    off0 = pl.multiple_of(base, 8)
    pltpu.sync_copy(src_hbm.at[pl.ds(off0, EPW)], sbuf)
    pltpu.sync_copy(dst_hbm.at[pl.ds(off0, EPW)], dbuf)

    def step(k, carry):
        gcp = []
        for s in range(NRING):
            rl, rr, sem = bufs[s]
            loc = pl.multiple_of((k * NRING + s) * KG, 8)
            off = pl.multiple_of(base + (k * NRING + s) * KG, 8)
            c0 = pltpu.async_copy(xl_hbm.at[sbuf.at[pl.ds(loc, KG)]], rl, sem)
            c1 = pltpu.async_copy(xr_hbm.at[dbuf.at[pl.ds(loc, KG)]], rr, sem)
            gcp.append((c0, c1, off))
        wcp = []
        for s in range(NRING):
            rl, rr, sem = bufs[s]
            c0, c1, off = gcp[s]
            c0.wait()
            c1.wait()
            w0 = pltpu.async_copy(rl, gs_hbm.at[pl.ds(off, KG), :], sem)
            w1 = pltpu.async_copy(rr, gd_hbm.at[pl.ds(off, KG), :], sem)
            wcp.append((w0, w1))
        for w0, w1 in wcp:
            w0.wait()
            w1.wait()
        return carry

    lax.fori_loop(0, EPW // KG // NRING, step, 0)


def _sc_gather(xl, xr, src, dst):
    ring = [
        pltpu.VMEM((EPW,), jnp.int32),
        pltpu.VMEM((EPW,), jnp.int32),
    ]
    for _ in range(NRING):
        ring += [
            pltpu.VMEM((KG, F), jnp.float32),
            pltpu.VMEM((KG, F), jnp.float32),
            pltpu.SemaphoreType.DMA,
        ]
    kfn = pl.kernel(
        _sc_gather_body,
        out_type=[
            jax.ShapeDtypeStruct((E, F), jnp.float32),
            jax.ShapeDtypeStruct((E, F), jnp.float32),
        ],
        mesh=_mesh(),
        compiler_params=pltpu.CompilerParams(needs_layout_passes=False),
        scratch_types=ring,
    )
    return kfn(xl, xr, src, dst)


def _sc_denom_body(dst_hbm, ex_hbm, part_hbm, acc, dbuf, ebuf):
    wid = lax.axis_index("s") * NC + lax.axis_index("c")

    def zero(k, carry):
        acc[pl.ds(pl.multiple_of(k * 16, 8), 16)] = jnp.zeros((16,), jnp.float32)
        return carry

    lax.fori_loop(0, NACC // 16, zero, 0)

    off0 = pl.multiple_of(wid * EPAD, 128)
    pltpu.sync_copy(dst_hbm.at[pl.ds(off0, EPAD)], dbuf)
    pltpu.sync_copy(ex_hbm.at[:, pl.ds(off0, EPAD)], ebuf)

    def step(i, carry):
        off = pl.multiple_of(i * 16, 8)
        dvec = dbuf[pl.ds(off, 16)]
        fours = dvec * H
        for h in range(H):
            plsc.addupdate_scatter(acc, [fours + h], ebuf[h, pl.ds(off, 16)])
        return carry

    lax.fori_loop(0, EPAD // 16, step, 0)
    pltpu.sync_copy(acc, part_hbm.at[wid])


def _sc_denom(dstp, ex4p):
    kfn = pl.kernel(
        _sc_denom_body,
        out_type=jax.ShapeDtypeStruct((NW, NACC), jnp.float32),
        mesh=_mesh(),
        compiler_params=pltpu.CompilerParams(needs_layout_passes=False),
        scratch_types=[
            pltpu.VMEM((NACC,), jnp.float32),
            pltpu.VMEM((EPAD,), jnp.int32),
            pltpu.VMEM((H, EPAD), jnp.float32),
        ],
    )
    return kfn(dstp, ex4p)


def _sc_agg_body(v4_hbm, dst_hbm, inv_hbm, a0_hbm, a1_hbm, y0_hbm, y1_hbm,
                 dbuf0, lbuf0, ibuf0, va0, vb0, vc0, vd0, ybuf0,
                 dbuf1, lbuf1, ibuf1, va1, vb1, vc1, vd1, ybuf1,
                 zbuf, agg_s, sem0, sem1):
    cid = lax.axis_index("c")
    sid = lax.axis_index("s")
    base = sid * EPS
    bufs = ((dbuf0, lbuf0, ibuf0, (va0, vb0, vc0, vd0), ybuf0, sem0),
            (dbuf1, lbuf1, ibuf1, (va1, vb1, vc1, vd1), ybuf1, sem1))

    def local_idx(lo, b):
        dbuf, lbuf = bufs[b][0], bufs[b][1]
        for q0 in (0, 16, 24):
            qs = pl.ds(q0, 16)
            dvec = dbuf[qs]
            inr = (dvec >= lo) & (dvec < lo + 5000)
            lbuf[qs] = jnp.where(inr, dvec - lo, 5000)

    def compute_y(b):
        ibuf, vbufs, ybuf = bufs[b][2], bufs[b][3], bufs[b][4]

        @plsc.parallel_loop(0, KV, step=1, unroll=4)
        def edge(e):
            wvec = ibuf[e, pl.ds(0, 16)]
            w0 = wvec[0]
            w1 = wvec[1]
            w2 = wvec[2]
            w3 = wvec[3]
            for q2 in range(CH // 16):
                sl = pl.ds(q2 * 16, 16)
                acc = w0 * vbufs[0][e, sl]
                acc = acc + w1 * vbufs[1][e, sl]
                acc = acc + w2 * vbufs[2][e, sl]
                acc = acc + w3 * vbufs[3][e, sl]
                ybuf[e, sl] = acc

    for r in range(2):
        lo = r * 5000

        def zrow(q, carry):
            for c in range(CH // 16):
                zbuf[q, pl.ds(c * 16, 16)] = jnp.zeros((16,), jnp.float32)
            return carry

        lax.fori_loop(0, 24, zrow, 0)
        for t in range(13):
            row0 = pl.multiple_of(sid * 312 + t * 24, 8)
            pltpu.sync_copy(zbuf, agg_s.at[pl.ds(row0, 24), :])

        @pl.when(sid == NS - 1)
        def _():
            pltpu.sync_copy(zbuf.at[pl.ds(0, 16), :],
                            agg_s.at[pl.ds(4992, 16), :])

        plsc.subcore_barrier()

        if r == 0:
            def load0(off, b):
                dbuf, ibuf, vbufs, sem = bufs[b][0], bufs[b][2], bufs[b][3], bufs[b][5]
                pltpu.sync_copy(dst_hbm.at[pl.ds(off, KV)], dbuf)
                ci = pltpu.async_copy(inv_hbm.at[dbuf], ibuf, sem)
                cvs = []

                @pl.when(cid == 0)
                def _():
                    for h in range(H):
                        pltpu.async_copy(
                            v4_hbm.at[pl.ds(off, KV), pl.ds(h * HD, CH)],
                            vbufs[h], sem)

                @pl.when(cid == 1)
                def _():
                    for h in range(H):
                        pltpu.async_copy(
                            v4_hbm.at[pl.ds(off, KV), pl.ds(h * HD + CH, CH)],
                            vbufs[h], sem)

                for h in range(H):
                    cvs.append(pltpu.make_async_copy(
                        v4_hbm.at[pl.ds(off, KV), pl.ds(h * HD, CH)],
                        vbufs[h], sem))
                return [ci] + cvs

            def fin0(off, b, cps):
                ybuf = bufs[b][4]
                for c in cps:
                    c.wait()
                local_idx(lo, b)
                compute_y(b)

                @pl.when(cid == 0)
                def _():
                    pltpu.sync_copy(ybuf, y0_hbm.at[pl.ds(off, KV), :])

                @pl.when(cid == 1)
                def _():
                    pltpu.sync_copy(ybuf, y1_hbm.at[pl.ds(off, KV), :])

                pltpu.sync_copy(ybuf, agg_s.at[bufs[b][1]], add=True)

            def step0(k, carry):
                offa = pl.multiple_of(base + (2 * k) * KV, 8)
                offb = pl.multiple_of(base + (2 * k + 1) * KV, 8)
                ca = load0(offa, 0)
                cb = load0(offb, 1)
                fin0(offa, 0, ca)
                fin0(offb, 1, cb)
                return carry

            lax.fori_loop(0, EPS // KV // 2, step0, 0)
        else:
            def load1(off, b):
                dbuf, _, _, _, ybuf, sem = bufs[b]
                pltpu.sync_copy(dst_hbm.at[pl.ds(off, KV)], dbuf)

                @pl.when(cid == 0)
                def _():
                    pltpu.async_copy(y0_hbm.at[pl.ds(off, KV), :], ybuf, sem)

                @pl.when(cid == 1)
                def _():
                    pltpu.async_copy(y1_hbm.at[pl.ds(off, KV), :], ybuf, sem)

                cy = pltpu.make_async_copy(y0_hbm.at[pl.ds(off, KV), :], ybuf,
                                           sem)
                return cy

            def fin1(off, b, cy):
                ybuf = bufs[b][4]
                cy.wait()
                local_idx(lo, b)
                pltpu.sync_copy(ybuf, agg_s.at[bufs[b][1]], add=True)

            def step1(k, carry):
                offa = pl.multiple_of(base + (2 * k) * KV, 8)
                offb = pl.multiple_of(base + (2 * k + 1) * KV, 8)
                ca = load1(offa, 0)
                cb = load1(offb, 1)
                fin1(offa, 0, ca)
                fin1(offb, 1, cb)
                return carry

            lax.fori_loop(0, EPS // KV // 2, step1, 0)

        plsc.subcore_barrier()

        for t in range(3):
            loc0 = pl.multiple_of(sid * 312 + t * 104, 8)
            glb0 = pl.multiple_of(r * 5000 + sid * 312 + t * 104, 8)

            @pl.when(cid == 0)
            def _():
                pltpu.sync_copy(agg_s.at[pl.ds(loc0, 104), :],
                                a0_hbm.at[pl.ds(glb0, 104), :])

            @pl.when(cid == 1)
            def _():
                pltpu.sync_copy(agg_s.at[pl.ds(loc0, 104), :],
                                a1_hbm.at[pl.ds(glb0, 104), :])

        @pl.when(sid == NS - 1)
        def _():
            glb1 = pl.multiple_of(r * 5000 + 4992, 8)

            @pl.when(cid == 0)
            def _():
                pltpu.sync_copy(agg_s.at[pl.ds(4992, 16), :],
                                a0_hbm.at[pl.ds(glb1, 16), :])

            @pl.when(cid == 1)
            def _():
                pltpu.sync_copy(agg_s.at[pl.ds(4992, 16), :],
                                a1_hbm.at[pl.ds(glb1, 16), :])


def _sc_aggregate(v4, dst, inv2):
    dbl = lambda: [
        pltpu.VMEM((KV,), jnp.int32),
        pltpu.VMEM((KV,), jnp.int32),
        pltpu.VMEM((KV, 128), jnp.float32),
        pltpu.VMEM((KV, CH), jnp.float32),
        pltpu.VMEM((KV, CH), jnp.float32),
        pltpu.VMEM((KV, CH), jnp.float32),
        pltpu.VMEM((KV, CH), jnp.float32),
        pltpu.VMEM((KV, CH), jnp.float32),
    ]
    kfn = pl.kernel(
        _sc_agg_body,
        out_type=[
            jax.ShapeDtypeStruct((N, CH), jnp.float32),
            jax.ShapeDtypeStruct((N, CH), jnp.float32),
            jax.ShapeDtypeStruct((E, CH), jnp.float32),
            jax.ShapeDtypeStruct((E, CH), jnp.float32),
        ],
        mesh=_mesh(),
        compiler_params=pltpu.CompilerParams(needs_layout_passes=False),
        scratch_types=dbl() + dbl() + [
            pltpu.VMEM((24, CH), jnp.float32),
            pltpu.VMEM_SHARED((5008, CH), jnp.float32),
            pltpu.SemaphoreType.DMA,
            pltpu.SemaphoreType.DMA,
        ],
    )
    a0, a1, _, _ = kfn(v4, dst, inv2)
    return a0, a1


def _layer(x2, src, dst, ea, wl, wr, we, attf, b, g, bb, pw):
    xl, xr = _mm(x2, wl, wr)
    gs, gd = _sc_gather(xl, xr, src, dst)
    ex4, v = _edge(gs, gd, ea, we, attf)
    dstp = jnp.pad(dst, (0, EPADL - E))
    ex4p = jnp.pad(ex4, ((0, 0), (0, EPADL - E)))
    part = _sc_denom(dstp, ex4p)
    inv = _inv(part)
    inv2 = jnp.pad(inv.reshape(N, H), ((0, 0), (0, 128 - H)))
    a0, a1 = _sc_aggregate(v, dst, inv2)
    return _post(a0, a1, b, g, bb, pw)


def kernel(x, edge_index, edge_attr, Wl1, Wr1, We1, att1, b1, bn1_g, bn1_b,
           Wl2, Wr2, We2, att2, b2, bn2_g, bn2_b, prelu_w):
    src = edge_index[0]
    dst = edge_index[1]
    r = lambda a: a.reshape(1, -1)
    h = _layer(x, src, dst, edge_attr, Wl1, Wr1, We1, r(att1), r(b1),
               r(bn1_g), r(bn1_b), None)
    h = _layer(h, src, dst, edge_attr, Wl2, Wr2, We2, r(att2), r(b2),
               r(bn2_g), r(bn2_b), r(prelu_w))
    return h


# speedup vs baseline: 9.0032x; 1.0342x over previous
import functools

import jax
import jax.numpy as jnp
from jax import lax
from jax.experimental import pallas as pl
from jax.experimental.pallas import tpu as pltpu
from jax.experimental.pallas import tpu_sc as plsc

N = 10000
E = 160000
F = 1024
H = 4
HD = 256
EDIM = 16
NC = 2
NS = 16
NW = NC * NS
EPW = E // NW
EPS = E // NS
EPAD = 5120
EPADL = NW * EPAD
KG = 8
NRING = 5
KV = 40
NACC = N * H
FH = F // 2
CH = HD // 2
BR = 312


def _mesh():
    return plsc.VectorSubcoreMesh(
        core_axis_name="c", subcore_axis_name="s", num_cores=NC, num_subcores=NS)


def _mm_body(x_ref, wl_ref, wr_ref, xl_ref, xr_ref):
    x = x_ref[...]
    xl_ref[...] = jnp.dot(x, wl_ref[...], preferred_element_type=jnp.float32)
    xr_ref[...] = jnp.dot(x, wr_ref[...], preferred_element_type=jnp.float32)


def _mm(x, wl, wr):
    n, d = x.shape
    bn = 1000
    return pl.pallas_call(
        _mm_body,
        grid=(n // bn,),
        in_specs=[
            pl.BlockSpec((bn, d), lambda i: (i, 0)),
            pl.BlockSpec((d, F), lambda i: (0, 0)),
            pl.BlockSpec((d, F), lambda i: (0, 0)),
        ],
        out_specs=[
            pl.BlockSpec((bn, F), lambda i: (i, 0)),
            pl.BlockSpec((bn, F), lambda i: (i, 0)),
        ],
        out_shape=[
            jax.ShapeDtypeStruct((n, F), jnp.float32),
            jax.ShapeDtypeStruct((n, F), jnp.float32),
        ],
    )(x, wl, wr)


def _edge_body(gs_ref, gd_ref, ea_ref, we_ref, att_ref, ex_ref, v_ref):
    gs = gs_ref[...]
    u = gs + gd_ref[...] + jnp.dot(ea_ref[...], we_ref[...],
                                   preferred_element_type=jnp.float32)
    m = jnp.maximum(u, 0.0) + 0.2 * jnp.minimum(u, 0.0)
    am = m * att_ref[...]
    for h in range(H):
        sl = slice(h * HD, (h + 1) * HD)
        lg = jnp.sum(am[:, sl], axis=1)
        exh = jnp.exp(lg)
        ex_ref[h, :] = exh
        v_ref[:, sl] = gs[:, sl] * exh[:, None]


def _edge(gs, gd, ea, we, attf):
    be = 1280
    return pl.pallas_call(
        _edge_body,
        grid=(E // be,),
        in_specs=[
            pl.BlockSpec((be, F), lambda i: (i, 0)),
            pl.BlockSpec((be, F), lambda i: (i, 0)),
            pl.BlockSpec((be, EDIM), lambda i: (i, 0)),
            pl.BlockSpec((EDIM, F), lambda i: (0, 0)),
            pl.BlockSpec((1, F), lambda i: (0, 0)),
        ],
        out_specs=[
            pl.BlockSpec((H, be), lambda i: (0, i)),
            pl.BlockSpec((be, F), lambda i: (i, 0)),
        ],
        out_shape=[
            jax.ShapeDtypeStruct((H, E), jnp.float32),
            jax.ShapeDtypeStruct((E, F), jnp.float32),
        ],
    )(gs, gd, ea, we, attf)


def _inv_body(part_ref, inv_ref):
    s = jnp.sum(part_ref[...], axis=0)
    inv_ref[...] = 1.0 / (s + 1e-16)


def _inv(part):
    return pl.pallas_call(
        _inv_body,
        out_shape=jax.ShapeDtypeStruct((NACC,), jnp.float32),
    )(part)


def _post(a0, a1, b, g, bb, pw):
    def body(*refs):
        if pw is None:
            a0_ref, a1_ref, b_ref, g_ref, bb_ref, out_ref = refs
        else:
            a0_ref, a1_ref, b_ref, g_ref, bb_ref, pw_ref, out_ref = refs
        h = (jnp.concatenate([a0_ref[...], a1_ref[...]], axis=1) * 0.25
             + b_ref[...])
        mu = jnp.mean(h, axis=0, keepdims=True)
        d = h - mu
        var = jnp.mean(d * d, axis=0, keepdims=True)
        hn = d * lax.rsqrt(var + 1e-5) * g_ref[...] + bb_ref[...]
        if pw is None:
            out_ref[...] = hn
        else:
            out_ref[...] = (jnp.maximum(hn, 0.0)
                            + pw_ref[...] * jnp.minimum(hn, 0.0))

    args = (a0, a1, b, g, bb) if pw is None else (a0, a1, b, g, bb, pw)
    return pl.pallas_call(
        body,
        out_shape=jax.ShapeDtypeStruct((N, HD), jnp.float32),
    )(*args)


def _sc_gather_body(xl_hbm, xr_hbm, src_hbm, dst_hbm, gs_hbm, gd_hbm,
                    *scr):
    wid = lax.axis_index("s") * NC + lax.axis_index("c")
    base = wid * EPW
    sbuf, dbuf = scr[0], scr[1]
    bufs = tuple(scr[2 + 3 * s:2 + 3 * s + 3] for s in range(NRING))

    off0 = pl.multiple_of(base, 8)
    pltpu.sync_copy(src_hbm.at[pl.ds(off0, EPW)], sbuf)
    pltpu.sync_copy(dst_hbm.at[pl.ds(off0, EPW)], dbuf)

    def step(k, carry):
        gcp = []
        for s in range(NRING):
            rl, rr, sem = bufs[s]
            loc = pl.multiple_of((k * NRING + s) * KG, 8)
            off = pl.multiple_of(base + (k * NRING + s) * KG, 8)
            c0 = pltpu.async_copy(xl_hbm.at[sbuf.at[pl.ds(loc, KG)]], rl, sem)
            c1 = pltpu.async_copy(xr_hbm.at[dbuf.at[pl.ds(loc, KG)]], rr, sem)
            gcp.append((c0, c1, off))
        wcp = []
        for s in range(NRING):
            rl, rr, sem = bufs[s]
            c0, c1, off = gcp[s]
            c0.wait()
            c1.wait()
            w0 = pltpu.async_copy(rl, gs_hbm.at[pl.ds(off, KG), :], sem)
            w1 = pltpu.async_copy(rr, gd_hbm.at[pl.ds(off, KG), :], sem)
            wcp.append((w0, w1))
        for w0, w1 in wcp:
            w0.wait()
            w1.wait()
        return carry

    lax.fori_loop(0, EPW // KG // NRING, step, 0)


def _sc_gather(xl, xr, src, dst):
    ring = [
        pltpu.VMEM((EPW,), jnp.int32),
        pltpu.VMEM((EPW,), jnp.int32),
    ]
    for _ in range(NRING):
        ring += [
            pltpu.VMEM((KG, F), jnp.float32),
            pltpu.VMEM((KG, F), jnp.float32),
            pltpu.SemaphoreType.DMA,
        ]
    kfn = pl.kernel(
        _sc_gather_body,
        out_type=[
            jax.ShapeDtypeStruct((E, F), jnp.float32),
            jax.ShapeDtypeStruct((E, F), jnp.float32),
        ],
        mesh=_mesh(),
        compiler_params=pltpu.CompilerParams(needs_layout_passes=False),
        scratch_types=ring,
    )
    return kfn(xl, xr, src, dst)


def _sc_denom_body(dst_hbm, ex_hbm, part_hbm, acc, dbuf, ebuf):
    wid = lax.axis_index("s") * NC + lax.axis_index("c")

    def zero(k, carry):
        acc[pl.ds(pl.multiple_of(k * 16, 8), 16)] = jnp.zeros((16,), jnp.float32)
        return carry

    lax.fori_loop(0, NACC // 16, zero, 0)

    off0 = pl.multiple_of(wid * EPAD, 128)
    pltpu.sync_copy(dst_hbm.at[pl.ds(off0, EPAD)], dbuf)
    pltpu.sync_copy(ex_hbm.at[:, pl.ds(off0, EPAD)], ebuf)

    def step(i, carry):
        off = pl.multiple_of(i * 16, 8)
        dvec = dbuf[pl.ds(off, 16)]
        fours = dvec * H
        for h in range(H):
            plsc.addupdate_scatter(acc, [fours + h], ebuf[h, pl.ds(off, 16)])
        return carry

    lax.fori_loop(0, EPAD // 16, step, 0)
    pltpu.sync_copy(acc, part_hbm.at[wid])


def _sc_denom(dstp, ex4p):
    kfn = pl.kernel(
        _sc_denom_body,
        out_type=jax.ShapeDtypeStruct((NW, NACC), jnp.float32),
        mesh=_mesh(),
        compiler_params=pltpu.CompilerParams(needs_layout_passes=False),
        scratch_types=[
            pltpu.VMEM((NACC,), jnp.float32),
            pltpu.VMEM((EPAD,), jnp.int32),
            pltpu.VMEM((H, EPAD), jnp.float32),
        ],
    )
    return kfn(dstp, ex4p)


def _sc_agg_body(v4_hbm, dst_hbm, inv_hbm, a0_hbm, a1_hbm, y0_hbm, y1_hbm,
                 dstall, lbuf0, ibuf0, va0, vb0, vc0, vd0, ybuf0,
                 lbuf1, ibuf1, va1, vb1, vc1, vd1, ybuf1,
                 zbuf, agg_s, sem0, sem1):
    cid = lax.axis_index("c")
    sid = lax.axis_index("s")
    base = sid * EPS
    bufs = ((dstall, lbuf0, ibuf0, (va0, vb0, vc0, vd0), ybuf0, sem0),
            (dstall, lbuf1, ibuf1, (va1, vb1, vc1, vd1), ybuf1, sem1))
    pltpu.sync_copy(dst_hbm.at[pl.ds(pl.multiple_of(base, 8), EPS)], dstall)

    def local_idx(lo, b, loc):
        lbuf = bufs[b][1]
        for q0 in (0, 16, 24):
            dvec = dstall[pl.ds(pl.multiple_of(loc + q0, 8), 16)]
            inr = (dvec >= lo) & (dvec < lo + 5000)
            lbuf[pl.ds(q0, 16)] = jnp.where(inr, dvec - lo, 5000)

    def compute_y(b):
        ibuf, vbufs, ybuf = bufs[b][2], bufs[b][3], bufs[b][4]

        @plsc.parallel_loop(0, KV, step=1, unroll=4)
        def edge(e):
            wvec = ibuf[e, pl.ds(0, 16)]
            w0 = wvec[0]
            w1 = wvec[1]
            w2 = wvec[2]
            w3 = wvec[3]
            for q2 in range(CH // 16):
                sl = pl.ds(q2 * 16, 16)
                acc = w0 * vbufs[0][e, sl]
                acc = acc + w1 * vbufs[1][e, sl]
                acc = acc + w2 * vbufs[2][e, sl]
                acc = acc + w3 * vbufs[3][e, sl]
                ybuf[e, sl] = acc

    for r in range(2):
        lo = r * 5000

        def zrow(q, carry):
            for c in range(CH // 16):
                zbuf[q, pl.ds(c * 16, 16)] = jnp.zeros((16,), jnp.float32)
            return carry

        lax.fori_loop(0, 24, zrow, 0)
        for t in range(13):
            row0 = pl.multiple_of(sid * 312 + t * 24, 8)
            pltpu.sync_copy(zbuf, agg_s.at[pl.ds(row0, 24), :])

        @pl.when(sid == NS - 1)
        def _():
            pltpu.sync_copy(zbuf.at[pl.ds(0, 16), :],
                            agg_s.at[pl.ds(4992, 16), :])

        plsc.subcore_barrier()

        if r == 0:
            def load0(off, loc, b):
                ibuf, vbufs, sem = bufs[b][2], bufs[b][3], bufs[b][5]
                ci = pltpu.async_copy(
                    inv_hbm.at[dstall.at[pl.ds(loc, KV)]], ibuf, sem)
                cvs = []

                @pl.when(cid == 0)
                def _():
                    for h in range(H):
                        pltpu.async_copy(
                            v4_hbm.at[pl.ds(off, KV), pl.ds(h * HD, CH)],
                            vbufs[h], sem)

                @pl.when(cid == 1)
                def _():
                    for h in range(H):
                        pltpu.async_copy(
                            v4_hbm.at[pl.ds(off, KV), pl.ds(h * HD + CH, CH)],
                            vbufs[h], sem)

                for h in range(H):
                    cvs.append(pltpu.make_async_copy(
                        v4_hbm.at[pl.ds(off, KV), pl.ds(h * HD, CH)],
                        vbufs[h], sem))
                return [ci] + cvs

            def fin0(off, loc, b, cps):
                ybuf = bufs[b][4]
                for c in cps:
                    c.wait()
                local_idx(lo, b, loc)
                compute_y(b)

                @pl.when(cid == 0)
                def _():
                    pltpu.sync_copy(ybuf, y0_hbm.at[pl.ds(off, KV), :])

                @pl.when(cid == 1)
                def _():
                    pltpu.sync_copy(ybuf, y1_hbm.at[pl.ds(off, KV), :])

                pltpu.sync_copy(ybuf, agg_s.at[bufs[b][1]], add=True)

            def step0(k, carry):
                loca = pl.multiple_of((2 * k) * KV, 8)
                locb = pl.multiple_of((2 * k + 1) * KV, 8)
                offa = pl.multiple_of(base + (2 * k) * KV, 8)
                offb = pl.multiple_of(base + (2 * k + 1) * KV, 8)
                ca = load0(offa, loca, 0)
                cb = load0(offb, locb, 1)
                fin0(offa, loca, 0, ca)
                fin0(offb, locb, 1, cb)
                return carry

            lax.fori_loop(0, EPS // KV // 2, step0, 0)
        else:
            def load1(off, b):
                ybuf, sem = bufs[b][4], bufs[b][5]

                @pl.when(cid == 0)
                def _():
                    pltpu.async_copy(y0_hbm.at[pl.ds(off, KV), :], ybuf, sem)

                @pl.when(cid == 1)
                def _():
                    pltpu.async_copy(y1_hbm.at[pl.ds(off, KV), :], ybuf, sem)

                cy = pltpu.make_async_copy(y0_hbm.at[pl.ds(off, KV), :], ybuf,
                                           sem)
                return cy

            def fin1(off, loc, b, cy):
                ybuf = bufs[b][4]
                cy.wait()
                local_idx(lo, b, loc)
                pltpu.sync_copy(ybuf, agg_s.at[bufs[b][1]], add=True)

            def step1(k, carry):
                loca = pl.multiple_of((2 * k) * KV, 8)
                locb = pl.multiple_of((2 * k + 1) * KV, 8)
                offa = pl.multiple_of(base + (2 * k) * KV, 8)
                offb = pl.multiple_of(base + (2 * k + 1) * KV, 8)
                ca = load1(offa, 0)
                cb = load1(offb, 1)
                fin1(offa, loca, 0, ca)
                fin1(offb, locb, 1, cb)
                return carry

            lax.fori_loop(0, EPS // KV // 2, step1, 0)

        plsc.subcore_barrier()

        for t in range(3):
            loc0 = pl.multiple_of(sid * 312 + t * 104, 8)
            glb0 = pl.multiple_of(r * 5000 + sid * 312 + t * 104, 8)

            @pl.when(cid == 0)
            def _():
                pltpu.sync_copy(agg_s.at[pl.ds(loc0, 104), :],
                                a0_hbm.at[pl.ds(glb0, 104), :])

            @pl.when(cid == 1)
            def _():
                pltpu.sync_copy(agg_s.at[pl.ds(loc0, 104), :],
                                a1_hbm.at[pl.ds(glb0, 104), :])

        @pl.when(sid == NS - 1)
        def _():
            glb1 = pl.multiple_of(r * 5000 + 4992, 8)

            @pl.when(cid == 0)
            def _():
                pltpu.sync_copy(agg_s.at[pl.ds(4992, 16), :],
                                a0_hbm.at[pl.ds(glb1, 16), :])

            @pl.when(cid == 1)
            def _():
                pltpu.sync_copy(agg_s.at[pl.ds(4992, 16), :],
                                a1_hbm.at[pl.ds(glb1, 16), :])


def _sc_aggregate(v4, dst, inv2):
    dbl = lambda: [
        pltpu.VMEM((KV,), jnp.int32),
        pltpu.VMEM((KV, 128), jnp.float32),
        pltpu.VMEM((KV, CH), jnp.float32),
        pltpu.VMEM((KV, CH), jnp.float32),
        pltpu.VMEM((KV, CH), jnp.float32),
        pltpu.VMEM((KV, CH), jnp.float32),
        pltpu.VMEM((KV, CH), jnp.float32),
    ]
    kfn = pl.kernel(
        _sc_agg_body,
        out_type=[
            jax.ShapeDtypeStruct((N, CH), jnp.float32),
            jax.ShapeDtypeStruct((N, CH), jnp.float32),
            jax.ShapeDtypeStruct((E, CH), jnp.float32),
            jax.ShapeDtypeStruct((E, CH), jnp.float32),
        ],
        mesh=_mesh(),
        compiler_params=pltpu.CompilerParams(needs_layout_passes=False),
        scratch_types=[pltpu.VMEM((EPS,), jnp.int32)] + dbl() + dbl() + [
            pltpu.VMEM((24, CH), jnp.float32),
            pltpu.VMEM_SHARED((5008, CH), jnp.float32),
            pltpu.SemaphoreType.DMA,
            pltpu.SemaphoreType.DMA,
        ],
    )
    a0, a1, _, _ = kfn(v4, dst, inv2)
    return a0, a1


def _layer(x2, src, dst, ea, wl, wr, we, attf, b, g, bb, pw):
    xl, xr = _mm(x2, wl, wr)
    gs, gd = _sc_gather(xl, xr, src, dst)
    ex4, v = _edge(gs, gd, ea, we, attf)
    dstp = jnp.pad(dst, (0, EPADL - E))
    ex4p = jnp.pad(ex4, ((0, 0), (0, EPADL - E)))
    part = _sc_denom(dstp, ex4p)
    inv = _inv(part)
    inv2 = jnp.pad(inv.reshape(N, H), ((0, 0), (0, 128 - H)))
    a0, a1 = _sc_aggregate(v, dst, inv2)
    return _post(a0, a1, b, g, bb, pw)


def kernel(x, edge_index, edge_attr, Wl1, Wr1, We1, att1, b1, bn1_g, bn1_b,
           Wl2, Wr2, We2, att2, b2, bn2_g, bn2_b, prelu_w):
    src = edge_index[0]
    dst = edge_index[1]
    r = lambda a: a.reshape(1, -1)
    h = _layer(x, src, dst, edge_attr, Wl1, Wr1, We1, r(att1), r(b1),
               r(bn1_g), r(bn1_b), None)
    h = _layer(h, src, dst, edge_attr, Wl2, Wr2, We2, r(att2), r(b2),
               r(bn2_g), r(bn2_b), r(prelu_w))
    return h
